# baseline (speedup 1.0000x reference)
import jax
import jax.numpy as jnp
from jax.experimental import pallas as pl
from jax.experimental.pallas import tpu as pltpu


def _pick_row_block(total_rows, max_rows):
    cap = min(max_rows, total_rows // 2 if total_rows >= 16 else total_rows)
    best = 0
    d = 8
    while d <= cap:
        if total_rows % d == 0:
            best = d
        d += 8
    return best if best else total_rows


def _upsample_kernel(uw_ref, x_ref, o_ref):
    x = x_ref[...]
    wide = jnp.dot(x, uw_ref[...], preferred_element_type=jnp.float32)
    o_ref[...] = pltpu.repeat(wide, 2, axis=1).astype(o_ref.dtype)


def kernel(x):
    N, C, H, W = x.shape
    dt = x.dtype
    R = N * C * H
    x2 = x.reshape(R, W)

    TR = _pick_row_block(R, 1024)
    nblk = R // TR

    uw = (jnp.arange(W)[:, None] == jnp.arange(2 * W)[None, :] // 2).astype(dt)

    flops = 2 * R * W * (2 * W)
    bytes_accessed = jnp.dtype(dt).itemsize * (R * W + 4 * R * W + 2 * W * W)

    out2 = pl.pallas_call(
        _upsample_kernel,
        out_shape=jax.ShapeDtypeStruct((R, 4 * W), dt),
        grid=(nblk,),
        in_specs=[
            pl.BlockSpec((W, 2 * W), lambda i: (0, 0)),
            pl.BlockSpec((TR, W), lambda i: (i, 0)),
        ],
        out_specs=pl.BlockSpec((TR, 4 * W), lambda i: (i, 0)),
        compiler_params=pltpu.CompilerParams(dimension_semantics=("parallel",)),
        cost_estimate=pl.CostEstimate(flops=flops, transcendentals=0,
                                      bytes_accessed=bytes_accessed),
    )(uw, x2)
    return out2.reshape(N, C, 2 * H, 2 * W)


# speedup vs baseline: 2.0623x; 2.0623x over previous
import jax
import jax.numpy as jnp
from jax.experimental import pallas as pl
from jax.experimental.pallas import tpu as pltpu


def _pick_row_block(total_rows, max_rows):
    cap = min(max_rows, total_rows // 2 if total_rows >= 16 else total_rows)
    best = 0
    d = 8
    while d <= cap:
        if total_rows % d == 0:
            best = d
        d += 8
    return best if best else total_rows


def _upsample_kernel(uw_ref, x_ref, o_ref):
    x = x_ref[...]
    wide = jnp.dot(x, uw_ref[...], preferred_element_type=jnp.float32)
    tr, w2 = wide.shape
    o_ref[...] = jnp.broadcast_to(wide[:, None, :], (tr, 2, w2)).reshape(
        2 * tr, w2).astype(o_ref.dtype)


def kernel(x):
    N, C, H, W = x.shape
    dt = x.dtype
    R = N * C * H
    x2 = x.reshape(R, W)

    TR = _pick_row_block(R, 1024)
    nblk = R // TR

    uw = (jnp.arange(W)[:, None] == jnp.arange(2 * W)[None, :] // 2).astype(dt)

    flops = 2 * R * W * (2 * W)
    bytes_accessed = jnp.dtype(dt).itemsize * (R * W + 4 * R * W + 2 * W * W)

    out2 = pl.pallas_call(
        _upsample_kernel,
        out_shape=jax.ShapeDtypeStruct((2 * R, 2 * W), dt),
        grid=(nblk,),
        in_specs=[
            pl.BlockSpec((W, 2 * W), lambda i: (0, 0)),
            pl.BlockSpec((TR, W), lambda i: (i, 0)),
        ],
        out_specs=pl.BlockSpec((2 * TR, 2 * W), lambda i: (i, 0)),
        compiler_params=pltpu.CompilerParams(dimension_semantics=("parallel",)),
        cost_estimate=pl.CostEstimate(flops=flops, transcendentals=0,
                                      bytes_accessed=bytes_accessed),
    )(uw, x2)
    return out2.reshape(N, C, 2 * H, 2 * W)


# speedup vs baseline: 2.5366x; 1.2300x over previous
import numpy as np

import jax
import jax.numpy as jnp
from jax.experimental import pallas as pl
from jax.experimental.pallas import tpu as pltpu


def _pick_row_block(total_rows, max_rows):
    cap = min(max_rows, total_rows // 2 if total_rows >= 16 else total_rows)
    best = 0
    d = 8
    while d <= cap:
        if total_rows % d == 0:
            best = d
        d += 8
    return best if best else total_rows


def _upsample_kernel(uw_ref, x_ref, o_ref):
    x = x_ref[...]
    xd = jnp.repeat(x, 2, axis=0)
    o_ref[...] = jnp.dot(xd, uw_ref[...],
                         preferred_element_type=jnp.float32).astype(o_ref.dtype)


def kernel(x):
    N, C, H, W = x.shape
    dt = x.dtype
    R = N * C * H
    x2 = x.reshape(R, W)

    TR = _pick_row_block(R, 1024)
    nblk = R // TR

    uw = (jnp.arange(W)[:, None] == jnp.arange(2 * W)[None, :] // 2).astype(dt)

    flops = 2 * R * W * (2 * W)
    bytes_accessed = jnp.dtype(dt).itemsize * (R * W + 4 * R * W + 2 * W * W)

    out2 = pl.pallas_call(
        _upsample_kernel,
        out_shape=jax.ShapeDtypeStruct((2 * R, 2 * W), dt),
        grid=(nblk,),
        in_specs=[
            pl.BlockSpec((W, 2 * W), lambda i: (0, 0)),
            pl.BlockSpec((TR, W), lambda i: (i, 0)),
        ],
        out_specs=pl.BlockSpec((2 * TR, 2 * W), lambda i: (i, 0)),
        compiler_params=pltpu.CompilerParams(
            dimension_semantics=("arbitrary",)),
        cost_estimate=pl.CostEstimate(flops=flops, transcendentals=0,
                                      bytes_accessed=bytes_accessed),
    )(uw, x2)
    return out2.reshape(N, C, 2 * H, 2 * W)


# speedup vs baseline: 3.2780x; 1.2923x over previous
import numpy as np

import jax
import jax.numpy as jnp
from jax.experimental import pallas as pl
from jax.experimental.pallas import tpu as pltpu


def _pick_row_block(total_rows, max_rows):
    cap = min(max_rows, total_rows // 2 if total_rows >= 16 else total_rows)
    best = 0
    d = 8
    while d <= cap:
        if total_rows % d == 0:
            best = d
        d += 8
    return best if best else total_rows


def _upsample_kernel(uw_ref, x_ref, o_ref):
    x = x_ref[...]
    xd = jnp.repeat(x, 2, axis=0)
    o_ref[...] = jnp.dot(xd, uw_ref[...],
                         preferred_element_type=jnp.float32).astype(o_ref.dtype)


def kernel(x):
    N, C, H, W = x.shape
    dt = x.dtype
    R = N * C * H
    x2 = x.reshape(R, W)

    TR = _pick_row_block(R, 2048)
    nblk = R // TR

    uw = (jnp.arange(W)[:, None] == jnp.arange(2 * W)[None, :] // 2).astype(dt)

    flops = 2 * R * W * (2 * W)
    bytes_accessed = jnp.dtype(dt).itemsize * (R * W + 4 * R * W + 2 * W * W)

    out2 = pl.pallas_call(
        _upsample_kernel,
        out_shape=jax.ShapeDtypeStruct((2 * R, 2 * W), dt),
        grid=(nblk,),
        in_specs=[
            pl.BlockSpec((W, 2 * W), lambda i: (0, 0)),
            pl.BlockSpec((TR, W), lambda i: (i, 0)),
        ],
        out_specs=pl.BlockSpec((2 * TR, 2 * W), lambda i: (i, 0)),
        compiler_params=pltpu.CompilerParams(
            dimension_semantics=("arbitrary",)),
        cost_estimate=pl.CostEstimate(flops=flops, transcendentals=0,
                                      bytes_accessed=bytes_accessed),
    )(uw, x2)
    return out2.reshape(N, C, 2 * H, 2 * W)


# speedup vs baseline: 3.7866x; 1.1551x over previous
import numpy as np

import jax
import jax.numpy as jnp
from jax.experimental import pallas as pl
from jax.experimental.pallas import tpu as pltpu


def _pick_row_block(total_rows, max_rows):
    cap = min(max_rows, total_rows // 2 if total_rows >= 16 else total_rows)
    best = 0
    d = 8
    while d <= cap:
        if total_rows % d == 0:
            best = d
        d += 8
    return best if best else total_rows


def _upsample_kernel(uw_ref, x_ref, o_ref):
    x = x_ref[...]
    xd = jnp.repeat(x, 2, axis=0)
    o_ref[...] = jnp.dot(xd, uw_ref[...],
                         preferred_element_type=jnp.float32).astype(o_ref.dtype)


def kernel(x):
    N, C, H, W = x.shape
    dt = x.dtype
    R = N * C * H
    x2 = x.reshape(R, W)

    TR = _pick_row_block(R, 4096)
    nblk = R // TR

    uw = (jnp.arange(W)[:, None] == jnp.arange(2 * W)[None, :] // 2).astype(dt)

    flops = 2 * R * W * (2 * W)
    bytes_accessed = jnp.dtype(dt).itemsize * (R * W + 4 * R * W + 2 * W * W)

    out2 = pl.pallas_call(
        _upsample_kernel,
        out_shape=jax.ShapeDtypeStruct((2 * R, 2 * W), dt),
        grid=(nblk,),
        in_specs=[
            pl.BlockSpec((W, 2 * W), lambda i: (0, 0)),
            pl.BlockSpec((TR, W), lambda i: (i, 0)),
        ],
        out_specs=pl.BlockSpec((2 * TR, 2 * W), lambda i: (i, 0)),
        compiler_params=pltpu.CompilerParams(
            dimension_semantics=("arbitrary",),
            vmem_limit_bytes=56 * 1024 * 1024),
        cost_estimate=pl.CostEstimate(flops=flops, transcendentals=0,
                                      bytes_accessed=bytes_accessed),
    )(uw, x2)
    return out2.reshape(N, C, 2 * H, 2 * W)


# speedup vs baseline: 3.8972x; 1.0292x over previous
import numpy as np

import jax
import jax.numpy as jnp
from jax.experimental import pallas as pl
from jax.experimental.pallas import tpu as pltpu


def _pick_row_block(total_rows, max_rows):
    cap = min(max_rows, total_rows // 2 if total_rows >= 16 else total_rows)
    best = 0
    d = 8
    while d <= cap:
        if total_rows % d == 0:
            best = d
        d += 8
    return best if best else total_rows


def _upsample_kernel(uw_ref, x_ref, o_ref):
    x = x_ref[...]
    xd = jnp.repeat(x, 2, axis=0)
    o_ref[...] = jnp.dot(xd, uw_ref[...],
                         preferred_element_type=jnp.float32).astype(o_ref.dtype)


def kernel(x):
    N, C, H, W = x.shape
    dt = x.dtype
    R = N * C * H
    x2 = x.reshape(R, W)

    TR = _pick_row_block(R, 8192)
    nblk = R // TR

    uw = (jnp.arange(W)[:, None] == jnp.arange(2 * W)[None, :] // 2).astype(dt)

    flops = 2 * R * W * (2 * W)
    bytes_accessed = jnp.dtype(dt).itemsize * (R * W + 4 * R * W + 2 * W * W)

    out2 = pl.pallas_call(
        _upsample_kernel,
        out_shape=jax.ShapeDtypeStruct((2 * R, 2 * W), dt),
        grid=(nblk,),
        in_specs=[
            pl.BlockSpec((W, 2 * W), lambda i: (0, 0)),
            pl.BlockSpec((TR, W), lambda i: (i, 0)),
        ],
        out_specs=pl.BlockSpec((2 * TR, 2 * W), lambda i: (i, 0)),
        compiler_params=pltpu.CompilerParams(
            dimension_semantics=("arbitrary",),
            vmem_limit_bytes=56 * 1024 * 1024),
        cost_estimate=pl.CostEstimate(flops=flops, transcendentals=0,
                                      bytes_accessed=bytes_accessed),
    )(uw, x2)
    return out2.reshape(N, C, 2 * H, 2 * W)
